# baseline (speedup 1.0000x reference)
import functools

import jax
import jax.numpy as jnp
from jax import lax
from jax.experimental import pallas as pl
from jax.experimental.pallas import tpu as pltpu
from jax.experimental.pallas import tpu_sc as plsc

N = 1048576
D = 64
_NC = 2
_NS = 16
_NW = _NC * _NS
_L = 16
_NT = N // 128
_TPW = _NT // _NW
_RPW = N // _NW
_TB = 4
_BPQ = _TPW // _TB
_NTURN = 8 * _BPQ
_SLOTS = 12
_NTRIPLE = (_NTURN - 2) // _SLOTS
_GSTAGE = 2048

_mesh = plsc.VectorSubcoreMesh(core_axis_name="c", subcore_axis_name="s")


@functools.partial(
    pl.kernel,
    mesh=_mesh,
    out_type=jax.ShapeDtypeStruct((8 * _NT, 8, 128), jnp.float32),
    scratch_types=[
        pltpu.VMEM((_SLOTS, _TB, 8, 128), jnp.float32),
        pltpu.VMEM((_RPW,), jnp.float32),
        pltpu.VMEM((_GSTAGE,), jnp.int32),
        pltpu.VMEM((128,), jnp.float32),
        pltpu.VMEM((2, D), jnp.float32),
    ] + [pltpu.SemaphoreType.DMA] * (2 * _SLOTS),
)
def _pv_kernel(delta_hbm, vold_hbm, g_hbm, out_hbm,
               buf, fsb, gst, dvb, vb, *sems):
    sin = sems[:_SLOTS]
    sout = sems[_SLOTS:]
    wid = lax.axis_index("c") * _NS + lax.axis_index("s")
    wt0 = wid * _TPW
    wrow0 = wid * _RPW

    pltpu.sync_copy(vold_hbm, vb)
    zeros = jnp.zeros((_L,), jnp.float32)
    for j in range(D // _L):
        dvb[pl.ds(_L * j, _L)] = (vb[0, pl.ds(_L * j, _L)]
                                  - vb[1, pl.ds(_L * j, _L)])
        dvb[pl.ds(D + _L * j, _L)] = zeros

    for k in range(_SLOTS - 2):
        in_copy_static = pltpu.make_async_copy(
            delta_hbm.at[pl.ds(wt0 + k * _TB, _TB)], buf.at[k], sin[k])
        in_copy_static.start()

    def stage(st, carry):
        pltpu.sync_copy(
            g_hbm.at[pl.ds(wrow0 + st * _GSTAGE, _GSTAGE)], gst)

        def vec(k, c2):
            gv = gst[pl.ds(k * _L, _L)]
            fsb[pl.ds(st * _GSTAGE + k * _L, _L)] = (
                1.0 - 2.0 * gv.astype(jnp.float32))
            return c2
        lax.fori_loop(0, _GSTAGE // _L, vec, 0)
        return carry
    lax.fori_loop(0, _RPW // _GSTAGE, stage, 0)

    def in_copy(slot, u):
        q = lax.div(u, _BPQ)
        b = lax.rem(u, _BPQ)
        base = q * _NT + wt0 + b * _TB
        return pltpu.make_async_copy(
            delta_hbm.at[pl.ds(base, _TB)], buf.at[slot], sin[slot])

    def out_copy(slot, u):
        q = lax.div(u, _BPQ)
        b = lax.rem(u, _BPQ)
        base = q * _NT + wt0 + b * _TB
        return pltpu.make_async_copy(
            buf.at[slot], out_hbm.at[pl.ds(base, _TB)], sout[slot])

    def compute_block(slot, u):
        q = lax.div(u, _BPQ)
        b = lax.rem(u, _BPQ)
        dvec = dvb[pl.ds(q * 8, _L)]
        msp = [dvec.at[jnp.full((_L,), r, jnp.int32)].get(
                   mode="promise_in_bounds") for r in range(8)]
        fs0 = b * (_TB * 128)

        def tile_body(tb, carry):
            for c in range(8):
                fs = fsb[pl.ds(fs0 + tb * 128 + c * _L, _L)]
                for r in range(8):
                    v = buf[slot, tb, r, pl.ds(c * _L, _L)]
                    buf[slot, tb, r, pl.ds(c * _L, _L)] = v * fs * msp[r]
            return carry
        lax.fori_loop(0, _TB, tile_body, 0)

    def turn(slot, u):
        @pl.when(u >= 2)
        def _wait_prev_out():
            out_copy((slot + _SLOTS - 2) % _SLOTS, u - 2).wait()

        @pl.when(u + _SLOTS - 2 < _NTURN)
        def _start_next_in():
            in_copy((slot + _SLOTS - 2) % _SLOTS, u + _SLOTS - 2).start()

        in_copy(slot, u).wait()
        compute_block(slot, u)
        out_copy(slot, u).start()

    def triple(p, carry):
        for sq in range(_SLOTS):
            turn(sq, _SLOTS * p + sq)
        return carry

    lax.fori_loop(0, _NTRIPLE, triple, 0)

    for u in range(_SLOTS * _NTRIPLE, _NTURN):
        turn(u % _SLOTS, u)
    for u in range(_NTURN - 2, _NTURN):
        out_copy(u % _SLOTS, u).wait()


def kernel(delta, v_old, G_idx):
    dv = delta.reshape(_NT, 128, 8, 8).transpose(2, 0, 3, 1)
    dv = dv.reshape(8 * _NT, 8, 128)
    ov = _pv_kernel(dv, v_old, G_idx.astype(jnp.int32))
    out = ov.reshape(8, _NT, 8, 128).transpose(1, 3, 0, 2)
    return out.reshape(N, D)


# speedup vs baseline: 1.1789x; 1.1789x over previous
import functools

import jax
import jax.numpy as jnp
from jax import lax
from jax.experimental import pallas as pl
from jax.experimental.pallas import tpu as pltpu
from jax.experimental.pallas import tpu_sc as plsc

N = 1048576
D = 64
_NC = 2
_NS = 16
_NW = _NC * _NS
_L = 16
_NT = N // 128
_TPW = _NT // _NW
_RPW = N // _NW
_TB = 8
_BPQ = _TPW // _TB
_NTURN = 8 * _BPQ
_SLOTS = 8
_NTRIPLE = (_NTURN - 2) // _SLOTS
_GSTAGE = 2048

_mesh = plsc.VectorSubcoreMesh(core_axis_name="c", subcore_axis_name="s")


@functools.partial(
    pl.kernel,
    mesh=_mesh,
    out_type=jax.ShapeDtypeStruct((8 * _NT, 8, 128), jnp.float32),
    scratch_types=[
        pltpu.VMEM((_SLOTS, _TB, 8, 128), jnp.float32),
        pltpu.VMEM((_RPW,), jnp.float32),
        pltpu.VMEM((_GSTAGE,), jnp.int32),
        pltpu.VMEM((128,), jnp.float32),
        pltpu.VMEM((2, D), jnp.float32),
    ] + [pltpu.SemaphoreType.DMA] * (2 * _SLOTS),
)
def _pv_kernel(delta_hbm, vold_hbm, g_hbm, out_hbm,
               buf, fsb, gst, dvb, vb, *sems):
    sin = sems[:_SLOTS]
    sout = sems[_SLOTS:]
    wid = lax.axis_index("c") * _NS + lax.axis_index("s")
    wt0 = wid * _TPW
    wrow0 = wid * _RPW

    pltpu.sync_copy(vold_hbm, vb)
    zeros = jnp.zeros((_L,), jnp.float32)
    for j in range(D // _L):
        dvb[pl.ds(_L * j, _L)] = (vb[0, pl.ds(_L * j, _L)]
                                  - vb[1, pl.ds(_L * j, _L)])
        dvb[pl.ds(D + _L * j, _L)] = zeros

    for k in range(_SLOTS - 2):
        in_copy_static = pltpu.make_async_copy(
            delta_hbm.at[pl.ds(wt0 + k * _TB, _TB)], buf.at[k], sin[k])
        in_copy_static.start()

    def stage(st, carry):
        pltpu.sync_copy(
            g_hbm.at[pl.ds(wrow0 + st * _GSTAGE, _GSTAGE)], gst)

        def vec(k, c2):
            gv = gst[pl.ds(k * _L, _L)]
            fsb[pl.ds(st * _GSTAGE + k * _L, _L)] = (
                1.0 - 2.0 * gv.astype(jnp.float32))
            return c2
        lax.fori_loop(0, _GSTAGE // _L, vec, 0)
        return carry
    lax.fori_loop(0, _RPW // _GSTAGE, stage, 0)

    def in_copy(slot, u):
        q = lax.div(u, _BPQ)
        b = lax.rem(u, _BPQ)
        base = q * _NT + wt0 + b * _TB
        return pltpu.make_async_copy(
            delta_hbm.at[pl.ds(base, _TB)], buf.at[slot], sin[slot])

    def out_copy(slot, u):
        q = lax.div(u, _BPQ)
        b = lax.rem(u, _BPQ)
        base = q * _NT + wt0 + b * _TB
        return pltpu.make_async_copy(
            buf.at[slot], out_hbm.at[pl.ds(base, _TB)], sout[slot])

    def compute_block(slot, u):
        q = lax.div(u, _BPQ)
        b = lax.rem(u, _BPQ)
        dvec = dvb[pl.ds(q * 8, _L)]
        msp = [dvec.at[jnp.full((_L,), r, jnp.int32)].get(
                   mode="promise_in_bounds") for r in range(8)]
        fs0 = b * (_TB * 128)

        def tile_body(tb, carry):
            for c in range(8):
                fs = fsb[pl.ds(fs0 + tb * 128 + c * _L, _L)]
                for r in range(8):
                    v = buf[slot, tb, r, pl.ds(c * _L, _L)]
                    buf[slot, tb, r, pl.ds(c * _L, _L)] = v * fs * msp[r]
            return carry
        lax.fori_loop(0, _TB, tile_body, 0)

    def turn(slot, u):
        @pl.when(u >= 2)
        def _wait_prev_out():
            out_copy((slot + _SLOTS - 2) % _SLOTS, u - 2).wait()

        @pl.when(u + _SLOTS - 2 < _NTURN)
        def _start_next_in():
            in_copy((slot + _SLOTS - 2) % _SLOTS, u + _SLOTS - 2).start()

        in_copy(slot, u).wait()
        compute_block(slot, u)
        out_copy(slot, u).start()

    def triple(p, carry):
        for sq in range(_SLOTS):
            turn(sq, _SLOTS * p + sq)
        return carry

    lax.fori_loop(0, _NTRIPLE, triple, 0)

    for u in range(_SLOTS * _NTRIPLE, _NTURN):
        turn(u % _SLOTS, u)
    for u in range(_NTURN - 2, _NTURN):
        out_copy(u % _SLOTS, u).wait()


def kernel(delta, v_old, G_idx):
    dv = delta.reshape(_NT, 128, 8, 8).transpose(2, 0, 3, 1)
    dv = dv.reshape(8 * _NT, 8, 128)
    ov = _pv_kernel(dv, v_old, G_idx.astype(jnp.int32))
    out = ov.reshape(8, _NT, 8, 128).transpose(1, 3, 0, 2)
    return out.reshape(N, D)


# speedup vs baseline: 1.2012x; 1.0189x over previous
import functools

import jax
import jax.numpy as jnp
from jax import lax
from jax.experimental import pallas as pl
from jax.experimental.pallas import tpu as pltpu
from jax.experimental.pallas import tpu_sc as plsc

N = 1048576
D = 64
_NC = 2
_NS = 16
_NW = _NC * _NS
_L = 16
_NT = N // 128
_TPW = _NT // _NW
_RPW = N // _NW
_TB = 8
_BPQ = _TPW // _TB
_NTURN = 8 * _BPQ
_SLOTS = 6
_NTRIPLE = (_NTURN - 2) // _SLOTS
_GSTAGE = 2048

_mesh = plsc.VectorSubcoreMesh(core_axis_name="c", subcore_axis_name="s")


@functools.partial(
    pl.kernel,
    mesh=_mesh,
    out_type=jax.ShapeDtypeStruct((8 * _NT, 8, 128), jnp.float32),
    scratch_types=[
        pltpu.VMEM((_SLOTS, _TB, 8, 128), jnp.float32),
        pltpu.VMEM((_RPW,), jnp.float32),
        pltpu.VMEM((_GSTAGE,), jnp.int32),
        pltpu.VMEM((128,), jnp.float32),
        pltpu.VMEM((2, D), jnp.float32),
    ] + [pltpu.SemaphoreType.DMA] * (2 * _SLOTS),
)
def _pv_kernel(delta_hbm, vold_hbm, g_hbm, out_hbm,
               buf, fsb, gst, dvb, vb, *sems):
    sin = sems[:_SLOTS]
    sout = sems[_SLOTS:]
    wid = lax.axis_index("c") * _NS + lax.axis_index("s")
    wt0 = wid * _TPW
    wrow0 = wid * _RPW

    pltpu.sync_copy(vold_hbm, vb)
    zeros = jnp.zeros((_L,), jnp.float32)
    for j in range(D // _L):
        dvb[pl.ds(_L * j, _L)] = (vb[0, pl.ds(_L * j, _L)]
                                  - vb[1, pl.ds(_L * j, _L)])
        dvb[pl.ds(D + _L * j, _L)] = zeros

    for k in range(_SLOTS - 2):
        in_copy_static = pltpu.make_async_copy(
            delta_hbm.at[pl.ds(wt0 + k * _TB, _TB)], buf.at[k], sin[k])
        in_copy_static.start()

    def stage(st, carry):
        pltpu.sync_copy(
            g_hbm.at[pl.ds(wrow0 + st * _GSTAGE, _GSTAGE)], gst)

        def vec(k, c2):
            gv = gst[pl.ds(k * _L, _L)]
            fsb[pl.ds(st * _GSTAGE + k * _L, _L)] = (
                1.0 - 2.0 * gv.astype(jnp.float32))
            return c2
        lax.fori_loop(0, _GSTAGE // _L, vec, 0)
        return carry
    lax.fori_loop(0, _RPW // _GSTAGE, stage, 0)

    def in_copy(slot, u):
        q = lax.div(u, _BPQ)
        b = lax.rem(u, _BPQ)
        base = q * _NT + wt0 + b * _TB
        return pltpu.make_async_copy(
            delta_hbm.at[pl.ds(base, _TB)], buf.at[slot], sin[slot])

    def out_copy(slot, u):
        q = lax.div(u, _BPQ)
        b = lax.rem(u, _BPQ)
        base = q * _NT + wt0 + b * _TB
        return pltpu.make_async_copy(
            buf.at[slot], out_hbm.at[pl.ds(base, _TB)], sout[slot])

    def compute_block(slot, u):
        q = lax.div(u, _BPQ)
        b = lax.rem(u, _BPQ)
        dvec = dvb[pl.ds(q * 8, _L)]
        msp = [dvec.at[jnp.full((_L,), r, jnp.int32)].get(
                   mode="promise_in_bounds") for r in range(8)]
        fs0 = b * (_TB * 128)

        def tile_body(tb, carry):
            for c in range(8):
                fs = fsb[pl.ds(fs0 + tb * 128 + c * _L, _L)]
                for r in range(8):
                    v = buf[slot, tb, r, pl.ds(c * _L, _L)]
                    buf[slot, tb, r, pl.ds(c * _L, _L)] = v * fs * msp[r]
            return carry
        lax.fori_loop(0, _TB, tile_body, 0)

    def turn(slot, u):
        @pl.when(u >= 2)
        def _wait_prev_out():
            out_copy((slot + _SLOTS - 2) % _SLOTS, u - 2).wait()

        @pl.when(u + _SLOTS - 2 < _NTURN)
        def _start_next_in():
            in_copy((slot + _SLOTS - 2) % _SLOTS, u + _SLOTS - 2).start()

        in_copy(slot, u).wait()
        compute_block(slot, u)
        out_copy(slot, u).start()

    def triple(p, carry):
        for sq in range(_SLOTS):
            turn(sq, _SLOTS * p + sq)
        return carry

    lax.fori_loop(0, _NTRIPLE, triple, 0)

    for u in range(_SLOTS * _NTRIPLE, _NTURN):
        turn(u % _SLOTS, u)
    for u in range(_NTURN - 2, _NTURN):
        out_copy(u % _SLOTS, u).wait()


def kernel(delta, v_old, G_idx):
    dv = delta.reshape(_NT, 128, 8, 8).transpose(2, 0, 3, 1)
    dv = dv.reshape(8 * _NT, 8, 128)
    ov = _pv_kernel(dv, v_old, G_idx.astype(jnp.int32))
    out = ov.reshape(8, _NT, 8, 128).transpose(1, 3, 0, 2)
    return out.reshape(N, D)
